# baseline (speedup 1.0000x reference)
import functools

import jax
import jax.numpy as jnp
from jax import lax
from jax.experimental import pallas as pl
from jax.experimental.pallas import tpu as pltpu
from jax.experimental.pallas import tpu_sc as plsc

B, L = 16384, 20
FEAT = 200
D_MODEL = 512
VPAD = 128

NC, NS = 2, 16
NW = NC * NS
TOTAL = B * L
BPW = TOTAL // NW
CHUNK = 40
NCHUNK = BPW // CHUNK
NRING = 4
NQUAD = NCHUNK // NRING

BB = 64
RGRID = B // BB


def _table_body(cbfv_ref, w_ref, b_ref, out_ref):
    acc = lax.dot_general(
        cbfv_ref[...], w_ref[...],
        dimension_numbers=(((1,), (1,)), ((), ())),
        preferred_element_type=jnp.float32,
    )
    out_ref[...] = acc + b_ref[...]


def _fuse_table(cbfv_pad, W, b2d):
    return pl.pallas_call(
        _table_body,
        out_shape=jax.ShapeDtypeStruct((VPAD, D_MODEL), jnp.float32),
    )(cbfv_pad, W, b2d)


@functools.cache
def _build_sc_gather():
    mesh = plsc.VectorSubcoreMesh(
        core_axis_name="c", subcore_axis_name="s", num_cores=NC, num_subcores=NS
    )
    return pl.kernel(
        _sc_gather_body,
        out_type=jax.ShapeDtypeStruct((TOTAL, D_MODEL), jnp.float32),
        mesh=mesh,
        scratch_types=[
            pltpu.VMEM((BPW,), jnp.int32),
            pltpu.VMEM((CHUNK, D_MODEL), jnp.float32),
            pltpu.VMEM((CHUNK, D_MODEL), jnp.float32),
            pltpu.VMEM((CHUNK, D_MODEL), jnp.float32),
            pltpu.VMEM((CHUNK, D_MODEL), jnp.float32),
            pltpu.SemaphoreType.DMA,
            pltpu.SemaphoreType.DMA,
            pltpu.SemaphoreType.DMA,
            pltpu.SemaphoreType.DMA,
            pltpu.SemaphoreType.DMA,
            pltpu.SemaphoreType.DMA,
            pltpu.SemaphoreType.DMA,
            pltpu.SemaphoreType.DMA,
        ],
    )


def _sc_gather_body(table_hbm, idx_hbm, out_hbm, idx_v,
                    b0, b1, b2, b3, g0, g1, g2, g3, w0, w1, w2, w3):
    wid = lax.axis_index("s") * NC + lax.axis_index("c")
    base = wid * BPW
    bufs = (b0, b1, b2, b3)
    gsems = (g0, g1, g2, g3)
    wsems = (w0, w1, w2, w3)

    pltpu.sync_copy(idx_hbm.at[pl.ds(base, BPW)], idx_v)
    tslice = table_hbm.at[pl.ds(wid * VPAD, VPAD)]

    def start_g(g, slot):
        pltpu.async_copy(
            tslice.at[idx_v.at[pl.ds(g * CHUNK, CHUNK)]], bufs[slot], gsems[slot]
        )

    def wait_g(slot):
        pltpu.make_async_copy(
            table_hbm.at[pl.ds(0, CHUNK)], bufs[slot], gsems[slot]
        ).wait()

    def start_w(g, slot):
        pltpu.async_copy(
            bufs[slot], out_hbm.at[pl.ds(base + g * CHUNK, CHUNK)], wsems[slot]
        )

    def wait_w(slot):
        pltpu.make_async_copy(
            table_hbm.at[pl.ds(0, CHUNK)], bufs[slot], wsems[slot]
        ).wait()

    for j in range(NRING):
        start_g(j, j)

    def quad(q, carry):
        g = NRING * q
        for j in range(NRING):
            wait_g(j)
            start_w(g + j, j)
        for j in range(NRING):
            wait_w(j)
            start_g(g + NRING + j, j)
        return carry

    lax.fori_loop(0, NQUAD - 1, quad, 0)

    g = NRING * (NQUAD - 1)
    for j in range(NRING):
        wait_g(j)
        start_w(g + j, j)
    for j in range(NRING):
        wait_w(j)


def _relayout_body(flat_ref, out_ref):
    out_ref[...] = flat_ref[...].reshape(BB, L, D_MODEL)


def _relayout(flat):
    return pl.pallas_call(
        _relayout_body,
        grid=(RGRID,),
        in_specs=[pl.BlockSpec((BB * L, D_MODEL), lambda i: (i, 0))],
        out_specs=pl.BlockSpec((BB, L, D_MODEL), lambda i: (i, 0, 0)),
        out_shape=jax.ShapeDtypeStruct((B, L, D_MODEL), jnp.float32),
    )(flat)


def kernel(src, cbfv, W, b):
    cbfv_pad = jnp.pad(cbfv, ((0, VPAD - cbfv.shape[0]), (0, 0)))
    table = _fuse_table(cbfv_pad, W, b.reshape(1, D_MODEL))
    idx = src.reshape(-1).astype(jnp.int32)
    table = jnp.tile(table, (NW, 1))
    flat = _build_sc_gather()(table, idx)
    return flat.reshape(B, L, D_MODEL)


# speedup vs baseline: 3.1849x; 3.1849x over previous
import functools

import jax
import jax.numpy as jnp
from jax import lax
from jax.experimental import pallas as pl
from jax.experimental.pallas import tpu as pltpu
from jax.experimental.pallas import tpu_sc as plsc

B, L = 16384, 20
FEAT = 200
D_MODEL = 512
VPAD = 128

NC, NS = 2, 16
NW = NC * NS
TOTAL = B * L
BPW = TOTAL // NW
CHUNK = 80
NCHUNK = BPW // CHUNK
NRING = 2
NQUAD = NCHUNK // NRING

BB = 64
RGRID = B // BB


def _table_body(cbfv_ref, w_ref, b_ref, out_ref):
    acc = lax.dot_general(
        cbfv_ref[...], w_ref[...],
        dimension_numbers=(((1,), (1,)), ((), ())),
        preferred_element_type=jnp.float32,
    )
    out_ref[...] = acc + b_ref[...]


def _fuse_table(cbfv_pad, W, b2d):
    return pl.pallas_call(
        _table_body,
        out_shape=jax.ShapeDtypeStruct((VPAD, D_MODEL), jnp.float32),
    )(cbfv_pad, W, b2d)


@functools.cache
def _build_sc_gather():
    mesh = plsc.VectorSubcoreMesh(
        core_axis_name="c", subcore_axis_name="s", num_cores=NC, num_subcores=NS
    )
    return pl.kernel(
        _sc_gather_body,
        out_type=jax.ShapeDtypeStruct((TOTAL, D_MODEL), jnp.float32),
        mesh=mesh,
        scratch_types=[
            pltpu.VMEM((BPW,), jnp.int32),
            pltpu.VMEM((CHUNK, D_MODEL), jnp.float32),
            pltpu.VMEM((CHUNK, D_MODEL), jnp.float32),
            pltpu.SemaphoreType.DMA,
            pltpu.SemaphoreType.DMA,
            pltpu.SemaphoreType.DMA,
            pltpu.SemaphoreType.DMA,
        ],
    )


def _sc_gather_body(table_hbm, idx_hbm, out_hbm, idx_v,
                    b0, b1, g0, g1, w0, w1):
    wid = lax.axis_index("s") * NC + lax.axis_index("c")
    base = wid * BPW
    bufs = (b0, b1)
    gsems = (g0, g1)
    wsems = (w0, w1)

    pltpu.sync_copy(idx_hbm.at[pl.ds(base, BPW)], idx_v)
    tslice = table_hbm.at[pl.ds(wid * VPAD, VPAD)]

    def start_g(g, slot):
        pltpu.async_copy(
            tslice.at[idx_v.at[pl.ds(g * CHUNK, CHUNK)]], bufs[slot], gsems[slot]
        )

    def wait_g(slot):
        pltpu.make_async_copy(
            table_hbm.at[pl.ds(0, CHUNK)], bufs[slot], gsems[slot]
        ).wait()

    def start_w(g, slot):
        pltpu.async_copy(
            bufs[slot], out_hbm.at[pl.ds(base + g * CHUNK, CHUNK)], wsems[slot]
        )

    def wait_w(slot):
        pltpu.make_async_copy(
            table_hbm.at[pl.ds(0, CHUNK)], bufs[slot], wsems[slot]
        ).wait()

    for j in range(NRING):
        start_g(j, j)

    def quad(q, carry):
        g = NRING * q
        for j in range(NRING):
            wait_g(j)
            start_w(g + j, j)
        for j in range(NRING):
            wait_w(j)
            start_g(g + NRING + j, j)
        return carry

    lax.fori_loop(0, NQUAD - 1, quad, 0)

    g = NRING * (NQUAD - 1)
    for j in range(NRING):
        wait_g(j)
        start_w(g + j, j)
    for j in range(NRING):
        wait_w(j)


def _relayout_body(flat_ref, out_ref):
    out_ref[...] = flat_ref[...].reshape(BB, L, D_MODEL)


def _relayout(flat):
    return pl.pallas_call(
        _relayout_body,
        grid=(RGRID,),
        in_specs=[pl.BlockSpec((BB * L, D_MODEL), lambda i: (i, 0))],
        out_specs=pl.BlockSpec((BB, L, D_MODEL), lambda i: (i, 0, 0)),
        out_shape=jax.ShapeDtypeStruct((B, L, D_MODEL), jnp.float32),
    )(flat)


def kernel(src, cbfv, W, b):
    cbfv_pad = jnp.pad(cbfv, ((0, VPAD - cbfv.shape[0]), (0, 0)))
    table = _fuse_table(cbfv_pad, W, b.reshape(1, D_MODEL))
    idx = src.astype(jnp.int32).T.reshape(-1)
    table = jnp.tile(table, (NW, 1))
    flat = _build_sc_gather()(table, idx)
    return flat.reshape(L, B, D_MODEL).transpose(1, 0, 2)
